# baseline (speedup 1.0000x reference)
import jax
import jax.numpy as jnp
from jax.experimental import pallas as pl

_B, _N, _C = 16, 1024, 1024
_R = 2048


def _body(x_ref, o_ref):
    rows = (
        jax.lax.broadcasted_iota(jnp.int32, (_R, _C), 0) & (_N - 1)
    ).astype(jnp.float32)
    odd = x_ref[:, pl.Slice(1, 8, 2), :]
    v = odd.reshape(_R, _C)
    o_ref[...] = jnp.where(v > 0.5, rows, 0.0)


def kernel(x):
    xt = jnp.transpose(
        x.reshape(_B * _N, _C // 128, 128, 2), (0, 1, 3, 2)
    ).reshape(_B * _N, 16, 128)
    out = pl.pallas_call(
        _body,
        grid=(_B * _N // _R,),
        in_specs=[pl.BlockSpec((_R, 16, 128), lambda j: (j, 0, 0))],
        out_specs=pl.BlockSpec((_R, _C), lambda j: (j, 0)),
        out_shape=jax.ShapeDtypeStruct((_B * _N, _C), jnp.float32),
    )(xt)
    return out.reshape(_B, _N, _C)
